# baseline (speedup 1.0000x reference)
import jax
import jax.numpy as jnp
from jax.experimental import pallas as pl


def _body(w_ref, o_ref):
    o_ref[...] = w_ref[...][None]


def kernel(tokens, W_pos):
    batch, seq_len = tokens.shape
    d_model = W_pos.shape[1]
    bs = 256
    grid = (seq_len // bs, batch)
    return pl.pallas_call(
        _body,
        grid=grid,
        in_specs=[pl.BlockSpec((bs, d_model), lambda s, b: (s, 0))],
        out_specs=pl.BlockSpec((1, bs, d_model), lambda s, b: (b, s, 0)),
        out_shape=jax.ShapeDtypeStruct((batch, seq_len, d_model), W_pos.dtype),
    )(W_pos)


# speedup vs baseline: 1.4527x; 1.4527x over previous
import jax
import jax.numpy as jnp
from jax.experimental import pallas as pl


def _make_body(batch, bs, d_model):
    def _body(w_ref, o_ref):
        o_ref[...] = jnp.broadcast_to(w_ref[...][None], (batch, bs, d_model))
    return _body


def kernel(tokens, W_pos):
    batch, seq_len = tokens.shape
    d_model = W_pos.shape[1]
    bs = 256
    grid = (seq_len // bs,)
    return pl.pallas_call(
        _make_body(batch, bs, d_model),
        grid=grid,
        in_specs=[pl.BlockSpec((bs, d_model), lambda s: (s, 0))],
        out_specs=pl.BlockSpec((batch, bs, d_model), lambda s: (0, s, 0)),
        out_shape=jax.ShapeDtypeStruct((batch, seq_len, d_model), W_pos.dtype),
    )(W_pos)


# speedup vs baseline: 1.5010x; 1.0332x over previous
import jax
import jax.numpy as jnp
from jax.experimental import pallas as pl


def _make_body(batch, bs, d_model):
    def _body(w_ref, o_ref):
        o_ref[...] = jnp.broadcast_to(w_ref[...][None], (batch, bs, d_model))
    return _body


def kernel(tokens, W_pos):
    batch, seq_len = tokens.shape
    d_model = W_pos.shape[1]
    bs = 512
    grid = (seq_len // bs,)
    return pl.pallas_call(
        _make_body(batch, bs, d_model),
        grid=grid,
        in_specs=[pl.BlockSpec((bs, d_model), lambda s: (s, 0))],
        out_specs=pl.BlockSpec((batch, bs, d_model), lambda s: (0, s, 0)),
        out_shape=jax.ShapeDtypeStruct((batch, seq_len, d_model), W_pos.dtype),
    )(W_pos)
